# baseline (speedup 1.0000x reference)
import functools

import jax
import jax.numpy as jnp
from jax import lax
from jax.experimental import pallas as pl
from jax.experimental.pallas import tpu as pltpu
from jax.experimental.pallas import tpu_sc as plsc

_MU = 3.5
_L = 16


@functools.lru_cache(maxsize=None)
def _build(B: int, K: int):
    info = plsc.get_sparse_core_info()
    nw = info.num_cores * info.num_subcores
    assert B % nw == 0
    bpw = B // nw
    n_chunks = bpw // _L
    mesh = plsc.VectorSubcoreMesh(core_axis_name="c", subcore_axis_name="s")

    @functools.partial(
        pl.kernel,
        mesh=mesh,
        out_type=jax.ShapeDtypeStruct((B,), jnp.float32),
        compiler_params=pltpu.CompilerParams(
            needs_layout_passes=False, use_tc_tiling_on_sc=False
        ),
        scratch_types=[
            pltpu.VMEM((bpw,), jnp.int32),
            pltpu.VMEM((bpw,), jnp.int32),
            pltpu.VMEM((bpw, K), jnp.float32),
            pltpu.VMEM((bpw, K), jnp.float32),
            pltpu.VMEM((bpw,), jnp.float32),
            pltpu.VMEM((bpw,), jnp.float32),
            pltpu.VMEM((bpw,), jnp.float32),
            pltpu.SemaphoreType.DMA,
            pltpu.SemaphoreType.DMA,
            pltpu.SemaphoreType.DMA,
            pltpu.SemaphoreType.DMA,
        ],
    )
    def fwd(uidx_hbm, iidx_hbm, p_hbm, q_hbm, bu_hbm, bi_hbm, out_hbm,
            uidx_v, iidx_v, p_rows, q_rows, bu_v, bi_v, out_v,
            sem_p, sem_q, sem_bu, sem_bi):
        wid = lax.axis_index("s") * info.num_cores + lax.axis_index("c")
        base = wid * bpw

        pltpu.sync_copy(uidx_hbm.at[pl.ds(base, bpw)], uidx_v)
        pltpu.sync_copy(iidx_hbm.at[pl.ds(base, bpw)], iidx_v)

        cp_p = pltpu.async_copy(p_hbm.at[uidx_v], p_rows, sem_p)
        cp_q = pltpu.async_copy(q_hbm.at[iidx_v], q_rows, sem_q)
        cp_bu = pltpu.async_copy(bu_hbm.at[uidx_v], bu_v, sem_bu)
        cp_bi = pltpu.async_copy(bi_hbm.at[iidx_v], bi_v, sem_bi)
        cp_p.wait()
        cp_q.wait()
        cp_bu.wait()
        cp_bi.wait()

        def chunk(c, carry):
            r0 = c * _L
            rows = r0 + lax.iota(jnp.int32, _L)
            acc = jnp.zeros((_L,), jnp.float32)
            for k in range(K):
                kv = jnp.full((_L,), k, jnp.int32)
                pc = plsc.load_gather(p_rows, [rows, kv])
                qc = plsc.load_gather(q_rows, [rows, kv])
                acc = acc + pc * qc
            out_v[pl.ds(r0, _L)] = (
                _MU + bu_v[pl.ds(r0, _L)] + bi_v[pl.ds(r0, _L)] + acc
            )
            return carry

        lax.fori_loop(0, n_chunks, chunk, 0)

        pltpu.sync_copy(out_v, out_hbm.at[pl.ds(base, bpw)])

    return fwd


def kernel(user_idx, item_idx, P, Q, b_u, b_i):
    B = user_idx.shape[0]
    K = P.shape[1]
    fwd = _build(B, K)
    return fwd(user_idx.astype(jnp.int32), item_idx.astype(jnp.int32),
               P, Q, b_u, b_i)
